# baseline (speedup 1.0000x reference)
import functools

import jax
import jax.numpy as jnp
from jax import lax
from jax.experimental import pallas as pl
from jax.experimental.pallas import tpu as pltpu
from jax.experimental.pallas import tpu_sc as plsc

D = 32
NT = 200
NB = 16384
TT = NT // 8
TB = NB // 128
NUM_CORES = 2
NUM_SUBCORES = 16
NUM_WORKERS = NUM_CORES * NUM_SUBCORES
QB = TB // NUM_WORKERS


@jax.jit
def _lookup(tok_p, weight):
    mesh = plsc.VectorSubcoreMesh(core_axis_name="c", subcore_axis_name="s")

    @functools.partial(
        pl.kernel,
        mesh=mesh,
        out_type=jax.ShapeDtypeStruct((NT, D // 8, TB, 8, 128), jnp.float32),
        scratch_types=[
            pltpu.VMEM((TT, 8, 128), jnp.int32),
            pltpu.VMEM((128, D), jnp.float32),
            pltpu.VMEM((128, D), jnp.float32),
            pltpu.VMEM((D, 128), jnp.float32),
            pltpu.SemaphoreType.DMA,
            pltpu.SemaphoreType.DMA,
        ],
        compiler_params=pltpu.CompilerParams(
            use_tc_tiling_on_sc=False, needs_layout_passes=False
        ),
    )
    def k(tok_hbm, w_hbm, out_hbm, tokb, rows0, rows1, out_t, sem0, sem1):
        rows = (rows0, rows1)
        sems = (sem0, sem1)
        wid = lax.axis_index("s") * NUM_CORES + lax.axis_index("c")
        iota = lax.iota(jnp.int32, 16)
        row_lo = iota
        row_hi = iota + 16

        def fire(t, s):
            tt = t // 8
            r = t % 8
            pltpu.async_copy(w_hbm.at[tokb.at[tt, r]], rows[s], sems[s])

        def wait(s):
            pltpu.make_async_copy(
                w_hbm.at[pl.ds(0, 128)], rows[s], sems[s]
            ).wait()

        def transpose(s):
            @pl.loop(0, 128, step=8)
            def _(c0):
                for dc in range(8):
                    c = c0 + dc
                    col = jnp.full((16,), c, jnp.int32)
                    v0 = rows[s][c, pl.ds(0, 16)]
                    plsc.store_scatter(out_t, [row_lo, col], v0)
                    v1 = rows[s][c, pl.ds(16, 16)]
                    plsc.store_scatter(out_t, [row_hi, col], v1)

        def write(t, tb):
            for tr in range(D // 8):
                pltpu.sync_copy(
                    out_t.at[pl.ds(tr * 8, 8)], out_hbm.at[t, tr, tb]
                )

        @pl.loop(0, QB)
        def _(q):
            tb = wid * QB + q
            pltpu.sync_copy(tok_hbm.at[:, tb], tokb)
            fire(0, 0)
            fire(1, 1)

            @pl.loop(0, NT - 2, step=2)
            def _(j):
                for s in range(2):
                    t = j + s
                    wait(s)
                    transpose(s)
                    fire(t + 2, s)
                    write(t, tb)

            for s in range(2):
                wait(s)
                transpose(s)
                write(NT - 2 + s, tb)

    return k(tok_p, weight)


def kernel(token_ids, weight):
    tok = token_ids.astype(jnp.int32)
    tok_p = jnp.transpose(tok).reshape(TT, 8, TB, 128).transpose(0, 2, 1, 3)
    out_p = _lookup(tok_p, weight)
    return (out_p.transpose(0, 1, 3, 2, 4)
            .reshape(NT, D, NB)
            .transpose(2, 0, 1))


# speedup vs baseline: 1.1086x; 1.1086x over previous
import functools

import jax
import jax.numpy as jnp
from jax import lax
from jax.experimental import pallas as pl
from jax.experimental.pallas import tpu as pltpu
from jax.experimental.pallas import tpu_sc as plsc

D = 32
NT = 200
NB = 16384
TT = NT // 8
TB = NB // 128
NUM_CORES = 2
NUM_SUBCORES = 16
NUM_WORKERS = NUM_CORES * NUM_SUBCORES
QB = TB // NUM_WORKERS
NS = 8


@jax.jit
def _lookup(tok_p, weight):
    mesh = plsc.VectorSubcoreMesh(core_axis_name="c", subcore_axis_name="s")

    @functools.partial(
        pl.kernel,
        mesh=mesh,
        out_type=jax.ShapeDtypeStruct((NT, D // 8, TB, 8, 128), jnp.float32),
        scratch_types=(
            [pltpu.VMEM((TT, 8, 128), jnp.int32)]
            + [pltpu.VMEM((128, D), jnp.float32)] * NS
            + [pltpu.VMEM((D, 128), jnp.float32)] * NS
            + [pltpu.SemaphoreType.DMA] * (2 * NS)
        ),
        compiler_params=pltpu.CompilerParams(
            use_tc_tiling_on_sc=False, needs_layout_passes=False
        ),
    )
    def k(tok_hbm, w_hbm, out_hbm, tokb, *bufs):
        rows = bufs[:NS]
        outt = bufs[NS:2 * NS]
        gsem = bufs[2 * NS:3 * NS]
        wsem = bufs[3 * NS:4 * NS]
        wid = lax.axis_index("s") * NUM_CORES + lax.axis_index("c")
        iota = lax.iota(jnp.int32, 16)
        row_lo = iota
        row_hi = iota + 16

        def fire(t, s):
            tt = t // 8
            r = t % 8
            pltpu.async_copy(w_hbm.at[tokb.at[tt, r]], rows[s], gsem[s])

        def wait_g(s):
            pltpu.make_async_copy(
                w_hbm.at[pl.ds(0, 128)], rows[s], gsem[s]
            ).wait()

        def transpose(s):
            @pl.loop(0, 128, step=8)
            def _(c0):
                for dc in range(8):
                    c = c0 + dc
                    col = jnp.full((16,), c, jnp.int32)
                    v0 = rows[s][c, pl.ds(0, 16)]
                    plsc.store_scatter(outt[s], [row_lo, col], v0)
                    v1 = rows[s][c, pl.ds(16, 16)]
                    plsc.store_scatter(outt[s], [row_hi, col], v1)

        def fire_w(t, tb, s):
            for tr in range(D // 8):
                pltpu.async_copy(
                    outt[s].at[pl.ds(tr * 8, 8)], out_hbm.at[t, tr, tb],
                    wsem[s],
                )

        def wait_w(s):
            for tr in range(D // 8):
                pltpu.make_async_copy(
                    outt[s].at[pl.ds(tr * 8, 8)], out_hbm.at[0, tr, 0],
                    wsem[s],
                ).wait()

        @pl.loop(0, QB)
        def _(q):
            tb = wid * QB + q
            pltpu.sync_copy(tok_hbm.at[:, tb], tokb)
            for s in range(NS):
                fire(s, s)

            @pl.loop(0, NT - NS, step=NS)
            def _(j):
                for s in range(NS):
                    t = j + s

                    @pl.when(j > 0)
                    def _():
                        wait_w(s)

                    wait_g(s)
                    transpose(s)
                    fire(t + NS, s)
                    fire_w(t, tb, s)

            for s in range(NS):
                wait_w(s)
                wait_g(s)
                transpose(s)
                fire_w(NT - NS + s, tb, s)
            for s in range(NS):
                wait_w(s)

    return k(tok_p, weight)


def kernel(token_ids, weight):
    tok = token_ids.astype(jnp.int32)
    tok_p = jnp.transpose(tok).reshape(TT, 8, TB, 128).transpose(0, 2, 1, 3)
    out_p = _lookup(tok_p, weight)
    return (out_p.transpose(0, 1, 3, 2, 4)
            .reshape(NT, D, NB)
            .transpose(2, 0, 1))


# speedup vs baseline: 1.1143x; 1.0052x over previous
import functools

import jax
import jax.numpy as jnp
from jax import lax
from jax.experimental import pallas as pl
from jax.experimental.pallas import tpu as pltpu
from jax.experimental.pallas import tpu_sc as plsc

D = 32
NT = 200
NB = 16384
TT = NT // 8
TB = NB // 128
NUM_CORES = 2
NUM_SUBCORES = 16
NUM_WORKERS = NUM_CORES * NUM_SUBCORES
QB = TB // NUM_WORKERS
NS = 8


@jax.jit
def _lookup(tok_p, weight):
    mesh = plsc.VectorSubcoreMesh(core_axis_name="c", subcore_axis_name="s")

    @functools.partial(
        pl.kernel,
        mesh=mesh,
        out_type=jax.ShapeDtypeStruct((NT, D // 8, TB, 8, 128), jnp.float32),
        scratch_types=[
            pltpu.VMEM((TT, 8, 128), jnp.int32),
            pltpu.VMEM((NS, 128, D), jnp.float32),
            pltpu.VMEM((NS, D, 128), jnp.float32),
            pltpu.SemaphoreType.DMA((NS,)),
            pltpu.SemaphoreType.DMA((NS,)),
        ],
        compiler_params=pltpu.CompilerParams(
            use_tc_tiling_on_sc=False, needs_layout_passes=False
        ),
    )
    def k(tok_hbm, w_hbm, out_hbm, tokb, rows, outt, gsem, wsem):
        wid = lax.axis_index("s") * NUM_CORES + lax.axis_index("c")
        iota = lax.iota(jnp.int32, 16)
        row_lo = iota
        row_hi = iota + 16

        def fire(t, s):
            tt = t // 8
            r = t % 8
            pltpu.async_copy(
                w_hbm.at[tokb.at[tt, r]], rows.at[s], gsem.at[s]
            )

        def wait_g(s):
            pltpu.make_async_copy(
                w_hbm.at[pl.ds(0, 128)], rows.at[s], gsem.at[s]
            ).wait()

        def transpose(s):
            sl = jnp.full((16,), s, jnp.int32)
            for c in range(128):
                col = jnp.full((16,), c, jnp.int32)
                v0 = rows[s, c, pl.ds(0, 16)]
                plsc.store_scatter(outt, [sl, row_lo, col], v0)
                v1 = rows[s, c, pl.ds(16, 16)]
                plsc.store_scatter(outt, [sl, row_hi, col], v1)

        def fire_w(t, tb, s):
            for tr in range(D // 8):
                pltpu.async_copy(
                    outt.at[s, pl.ds(tr * 8, 8)], out_hbm.at[t, tr, tb],
                    wsem.at[s],
                )

        def wait_w(s):
            for tr in range(D // 8):
                pltpu.make_async_copy(
                    outt.at[s, pl.ds(tr * 8, 8)], out_hbm.at[0, tr, 0],
                    wsem.at[s],
                ).wait()

        @pl.loop(0, QB)
        def _(q):
            tb = wid * QB + q
            pltpu.sync_copy(tok_hbm.at[:, tb], tokb)
            for s0 in range(NS):
                fire(s0, s0)

            @pl.loop(0, NT - NS)
            def _(t):
                s = t % NS

                @pl.when(t >= NS)
                def _():
                    wait_w(s)

                wait_g(s)
                transpose(s)
                fire(t + NS, s)
                fire_w(t, tb, s)

            @pl.loop(NT - NS, NT)
            def _(t):
                s = t % NS
                wait_w(s)
                wait_g(s)
                transpose(s)
                fire_w(t, tb, s)

            for s0 in range(NS):
                wait_w(s0)

    return k(tok_p, weight)


def kernel(token_ids, weight):
    tok = token_ids.astype(jnp.int32)
    tok_p = jnp.transpose(tok).reshape(TT, 8, TB, 128).transpose(0, 2, 1, 3)
    out_p = _lookup(tok_p, weight)
    return (out_p.transpose(0, 1, 3, 2, 4)
            .reshape(NT, D, NB)
            .transpose(2, 0, 1))


# speedup vs baseline: 1.6186x; 1.4525x over previous
import functools

import jax
import jax.numpy as jnp
from jax import lax
from jax.experimental import pallas as pl
from jax.experimental.pallas import tpu as pltpu
from jax.experimental.pallas import tpu_sc as plsc

D = 32
NT = 200
NB = 16384
TT = NT // 8
TB = NB // 128
NUM_CORES = 2
NUM_SUBCORES = 16
NUM_WORKERS = NUM_CORES * NUM_SUBCORES
QB = TB // NUM_WORKERS
NS = 8


@jax.jit
def _lookup(tok_p, weight):
    mesh = plsc.VectorSubcoreMesh(core_axis_name="c", subcore_axis_name="s")

    @functools.partial(
        pl.kernel,
        mesh=mesh,
        out_type=jax.ShapeDtypeStruct((NT, D // 8, TB, 8, 128), jnp.float32),
        scratch_types=[
            pltpu.VMEM((TT, 8, 128), jnp.int32),
            pltpu.VMEM((NS, 128, D), jnp.float32),
            pltpu.VMEM((2, D, 128), jnp.float32),
            pltpu.SemaphoreType.DMA((NS,)),
            pltpu.SemaphoreType.DMA((2,)),
        ],
        compiler_params=pltpu.CompilerParams(
            use_tc_tiling_on_sc=False, needs_layout_passes=False
        ),
    )
    def k(tok_hbm, w_hbm, out_hbm, tokb, rows, outt, gsem, wsem):
        wid = lax.axis_index("s") * NUM_CORES + lax.axis_index("c")
        iota = lax.iota(jnp.int32, 16)
        row_lo = iota
        row_hi = iota + 16

        def fire(t, s):
            tt = t // 8
            r = t % 8
            pltpu.async_copy(
                w_hbm.at[tokb.at[tt, r]], rows.at[s], gsem.at[s]
            )

        def wait_g(s):
            pltpu.make_async_copy(
                w_hbm.at[pl.ds(0, 128)], rows.at[s], gsem.at[s]
            ).wait()

        def transpose(s, w):
            sl = jnp.full((16,), s, jnp.int32)
            wl = jnp.full((16,), w, jnp.int32)
            for c0 in range(0, 128, 16):
                rowv = iota + c0
                for d0 in range(0, D, 16):
                    for j in range(16):
                        colv = ((iota + j) & 15) + d0
                        v = plsc.load_gather(rows, [sl, rowv, colv])
                        plsc.store_scatter(outt, [wl, colv, rowv], v)

        def fire_w(t, tb, w):
            for tr in range(D // 8):
                pltpu.async_copy(
                    outt.at[w, pl.ds(tr * 8, 8)], out_hbm.at[t, tr, tb],
                    wsem.at[w],
                )

        def wait_w(w):
            for tr in range(D // 8):
                pltpu.make_async_copy(
                    outt.at[w, pl.ds(tr * 8, 8)], out_hbm.at[0, tr, 0],
                    wsem.at[w],
                ).wait()

        @pl.loop(0, QB)
        def _(q):
            tb = wid * QB + q
            pltpu.sync_copy(tok_hbm.at[:, tb], tokb)
            for s0 in range(NS):
                fire(s0, s0)

            @pl.loop(0, NT)
            def _(t):
                s = t % NS
                w = t % 2

                @pl.when(t >= 2)
                def _():
                    wait_w(w)

                wait_g(s)
                transpose(s, w)

                @pl.when(t < NT - NS)
                def _():
                    fire(t + NS, s)

                fire_w(t, tb, w)

            for w0 in range(2):
                wait_w(w0)

    return k(tok_p, weight)


def kernel(token_ids, weight):
    tok = token_ids.astype(jnp.int32)
    tok_p = jnp.transpose(tok).reshape(TT, 8, TB, 128).transpose(0, 2, 1, 3)
    out_p = _lookup(tok_p, weight)
    return (out_p.transpose(0, 1, 3, 2, 4)
            .reshape(NT, D, NB)
            .transpose(2, 0, 1))
